# Initial kernel scaffold; baseline (speedup 1.0000x reference)
import jax
import jax.numpy as jnp
from jax.experimental import pallas as pl


def kernel(h, pos, edge_index, coord_diff, distances, edge_attr, W1, b1, W2, b2, Wc1, bc1, Wc2, bc2, Wc3, bc3):
    raise NotImplementedError("write your pallas kernel here")


# speedup vs baseline: 2.9156x; 2.9156x over previous
import functools

import jax
import jax.numpy as jnp
from jax import lax
from jax.experimental import pallas as pl
from jax.experimental.pallas import tpu as pltpu
from jax.experimental.pallas import tpu_sc as plsc

_NW = 32
_SEG = 128


def _k1_body(h_ref, wc1_ref, p_ref, q_ref):
    hb = h_ref[...]
    p_ref[...] = jnp.dot(hb, wc1_ref[0:128, :], preferred_element_type=jnp.float32)
    q_ref[...] = jnp.dot(hb, wc1_ref[128:256, :], preferred_element_type=jnp.float32)


def _project_pq(h, wc1):
    n, hdim = h.shape
    bn = 2000
    return pl.pallas_call(
        _k1_body,
        grid=(n // bn,),
        in_specs=[
            pl.BlockSpec((bn, hdim), lambda i: (i, 0)),
            pl.BlockSpec((3 * hdim, hdim), lambda i: (0, 0)),
        ],
        out_specs=[
            pl.BlockSpec((bn, hdim), lambda i: (i, 0)),
            pl.BlockSpec((bn, hdim), lambda i: (i, 0)),
        ],
        out_shape=[
            jax.ShapeDtypeStruct((n, hdim), jnp.float32),
            jax.ShapeDtypeStruct((n, hdim), jnp.float32),
        ],
    )(h, wc1)


def _gather_pq(p, q, ii3, jj3, e):
    n, hdim = p.shape
    segs_per_chunk = ii3.shape[1]
    ch = segs_per_chunk * _SEG
    nch = e // ch
    iters = (nch + _NW - 1) // _NW

    mesh = plsc.VectorSubcoreMesh(core_axis_name="c", subcore_axis_name="s", num_cores=2, num_subcores=16)

    def body(p_hbm, q_hbm, ii_hbm, jj_hbm, gp_hbm, gq_hbm, idxb, rows, sem):
        c = lax.axis_index("c")
        s = lax.axis_index("s")
        wid = s * 2 + c

        @pl.loop(0, iters)
        def _chunk(k):
            chunk = wid + _NW * k

            @pl.when(chunk < nch)
            def _():
                base_e = chunk * ch
                for tab_hbm, id_hbm, out_hbm in (
                    (p_hbm, ii_hbm, gp_hbm),
                    (q_hbm, jj_hbm, gq_hbm),
                ):
                    pltpu.sync_copy(id_hbm.at[chunk], idxb)
                    descs = [
                        pltpu.async_copy(
                            tab_hbm.at[idxb.at[j]],
                            rows.at[pl.ds(j * _SEG, _SEG)],
                            sem,
                        )
                        for j in range(segs_per_chunk)
                    ]
                    for d in descs:
                        d.wait()
                    pltpu.sync_copy(rows, out_hbm.at[pl.ds(base_e, ch)])

    k = pl.kernel(
        body,
        out_type=[
            jax.ShapeDtypeStruct((e, hdim), jnp.float32),
            jax.ShapeDtypeStruct((e, hdim), jnp.float32),
        ],
        mesh=mesh,
        scratch_types=[
            pltpu.VMEM((segs_per_chunk, _SEG), jnp.int32),
            pltpu.VMEM((ch, hdim), jnp.float32),
            pltpu.SemaphoreType.DMA,
        ],
    )
    return k(p, q, ii3, jj3)


def _k3_body(gp_ref, gq_ref, ea_ref, cd4_ref, wc1c_ref, bc1_ref, wc2_ref,
             bc2_ref, wc3_ref, bc3_ref, out_ref):
    pre = (gp_ref[...] + gq_ref[...]
           + jnp.dot(ea_ref[...], wc1c_ref[...], preferred_element_type=jnp.float32)
           + bc1_ref[...])
    x1 = pre * jax.nn.sigmoid(pre)
    pre2 = jnp.dot(x1, wc2_ref[...], preferred_element_type=jnp.float32) + bc2_ref[...]
    x2 = pre2 * jax.nn.sigmoid(pre2)
    phi = jnp.dot(x2, wc3_ref[...], preferred_element_type=jnp.float32) + bc3_ref[...]
    cd = cd4_ref[...]
    col = lax.broadcasted_iota(jnp.int32, cd.shape, 1)
    out_ref[...] = jnp.where(col < 3, cd * phi, cd)


def _edge_mlp(gp, gq, edge_attr, cd4, wc1c, bc1, wc2, bc2, wc3, bc3):
    e, hdim = gp.shape
    be = 2000
    return pl.pallas_call(
        _k3_body,
        grid=(e // be,),
        in_specs=[
            pl.BlockSpec((be, hdim), lambda i: (i, 0)),
            pl.BlockSpec((be, hdim), lambda i: (i, 0)),
            pl.BlockSpec((be, hdim), lambda i: (i, 0)),
            pl.BlockSpec((be, 4), lambda i: (i, 0)),
            pl.BlockSpec((hdim, hdim), lambda i: (0, 0)),
            pl.BlockSpec((1, hdim), lambda i: (0, 0)),
            pl.BlockSpec((hdim, hdim), lambda i: (0, 0)),
            pl.BlockSpec((1, hdim), lambda i: (0, 0)),
            pl.BlockSpec((hdim, 1), lambda i: (0, 0)),
            pl.BlockSpec((1, 1), lambda i: (0, 0)),
        ],
        out_specs=pl.BlockSpec((be, 4), lambda i: (i, 0)),
        out_shape=jax.ShapeDtypeStruct((e, 4), jnp.float32),
    )(gp, gq, edge_attr, cd4, wc1c, bc1, wc2, bc2, wc3, bc3)


_NPAD = 10240


def _scatter_td(tx, ty, tz, tdd, ii, zeros_np, n, e):
    ew = e // _NW
    ch = 2000
    nchunks = ew // ch
    comps = 4

    mesh = plsc.VectorSubcoreMesh(core_axis_name="c", subcore_axis_name="s", num_cores=2, num_subcores=16)

    def body(tx_hbm, ty_hbm, tz_hbm, td_hbm, ii_hbm, z_hbm, out_hbm,
             idxb, vbx, vby, vbz, vbd, accx, accy, accz, accd, sem):
        c = lax.axis_index("c")
        s = lax.axis_index("s")
        wid = s * 2 + c
        base_w = wid * ew
        accs = (accx, accy, accz, accd)
        vbs = (vbx, vby, vbz, vbd)
        srcs = (tx_hbm, ty_hbm, tz_hbm, td_hbm)

        for a in accs:
            pltpu.sync_copy(z_hbm, a)

        @pl.loop(0, nchunks)
        def _chunk(k):
            base = base_w + k * ch
            descs = [pltpu.async_copy(ii_hbm.at[pl.ds(base, ch)], idxb, sem)]
            descs += [
                pltpu.async_copy(srcs[ci].at[pl.ds(base, ch)], vbs[ci], sem)
                for ci in range(comps)
            ]
            for d in descs:
                d.wait()

            @pl.loop(0, ch // 16)
            def _vec(kk):
                iv = idxb[pl.ds(kk * 16, 16)]
                for ci in range(comps):
                    vv = vbs[ci][pl.ds(kk * 16, 16)]
                    plsc.addupdate_scatter(accs[ci], [iv], vv)

        for ci in range(comps):
            pltpu.sync_copy(
                accs[ci],
                out_hbm.at[pl.ds(wid * (comps * _NPAD) + ci * _NPAD, _NPAD)])

    k = pl.kernel(
        body,
        out_type=jax.ShapeDtypeStruct((_NW * comps * _NPAD,), jnp.float32),
        mesh=mesh,
        compiler_params=pltpu.CompilerParams(needs_layout_passes=False),
        scratch_types=[
            pltpu.VMEM((ch,), jnp.int32),
        ] + [pltpu.VMEM((ch,), jnp.float32) for _ in range(comps)]
          + [pltpu.VMEM((_NPAD,), jnp.float32) for _ in range(comps)]
          + [pltpu.SemaphoreType.DMA],
    )
    return k(tx, ty, tz, tdd, ii, zeros_np)


def _k4b_body(par_ref, out_ref):
    out_ref[...] = jnp.sum(par_ref[...], axis=0, keepdims=True)


def _reduce_partials(par2d):
    w, m = par2d.shape
    return pl.pallas_call(
        _k4b_body,
        grid=(1,),
        in_specs=[pl.BlockSpec((w, m), lambda i: (0, 0))],
        out_specs=pl.BlockSpec((1, m), lambda i: (0, 0)),
        out_shape=jax.ShapeDtypeStruct((1, m), jnp.float32),
    )(par2d)


def _k5_body(par_ref, pos_ref, h_ref, w1_ref, b1_ref, w2_ref, b2_ref,
             pos_out_ref, h_out_ref):
    agg = par_ref[...]
    pos_out_ref[...] = pos_ref[...] + agg[:, 0:3]
    d = agg[:, 3:4]
    y1p = d * w1_ref[...] + b1_ref[...]
    y1 = y1p * jax.nn.sigmoid(y1p)
    y = jnp.dot(y1, w2_ref[...], preferred_element_type=jnp.float32) + b2_ref[...]
    h_out_ref[...] = h_ref[...] + y


def _node_update(agg4, pos, h, w1, b1, w2, b2):
    n, hdim = h.shape
    bn = 2000
    return pl.pallas_call(
        _k5_body,
        grid=(n // bn,),
        in_specs=[
            pl.BlockSpec((bn, 4), lambda i: (i, 0)),
            pl.BlockSpec((bn, 3), lambda i: (i, 0)),
            pl.BlockSpec((bn, hdim), lambda i: (i, 0)),
            pl.BlockSpec((1, 16), lambda i: (0, 0)),
            pl.BlockSpec((1, 16), lambda i: (0, 0)),
            pl.BlockSpec((16, hdim), lambda i: (0, 0)),
            pl.BlockSpec((1, hdim), lambda i: (0, 0)),
        ],
        out_specs=[
            pl.BlockSpec((bn, 3), lambda i: (i, 0)),
            pl.BlockSpec((bn, hdim), lambda i: (i, 0)),
        ],
        out_shape=[
            jax.ShapeDtypeStruct((n, 3), jnp.float32),
            jax.ShapeDtypeStruct((n, hdim), jnp.float32),
        ],
    )(agg4, pos, h, w1, b1, w2, b2)


def kernel(h, pos, edge_index, coord_diff, distances, edge_attr,
           W1, b1, W2, b2, Wc1, bc1, Wc2, bc2, Wc3, bc3):
    n, hdim = h.shape
    e = edge_attr.shape[0]

    ii = edge_index[0]
    jj = edge_index[1]
    ii3g = ii.reshape(e // (4 * _SEG), 4, _SEG)
    jj3g = jj.reshape(e // (4 * _SEG), 4, _SEG)
    cd4 = jnp.concatenate([coord_diff, distances], axis=1)

    p, q = _project_pq(h, Wc1)
    gp, gq = _gather_pq(p, q, ii3g, jj3g, e)
    td4 = _edge_mlp(
        gp, gq, edge_attr, cd4,
        Wc1[2 * hdim:], bc1.reshape(1, hdim), Wc2, bc2.reshape(1, hdim),
        Wc3, bc3.reshape(1, 1),
    )
    tx, ty, tz, tdd = (td4[:, 0], td4[:, 1], td4[:, 2], td4[:, 3])
    zeros_np = jnp.zeros((_NPAD,), jnp.float32)
    partials = _scatter_td(tx, ty, tz, tdd, ii, zeros_np, n, e)
    summed = _reduce_partials(partials.reshape(_NW, 4 * _NPAD))
    agg4 = summed.reshape(4, _NPAD)[:, :n].T
    pos_out, h_out = _node_update(
        agg4, pos, h,
        W1, b1.reshape(1, 16), W2, b2.reshape(1, hdim),
    )
    return (pos_out, h_out)
